# Initial kernel scaffold; baseline (speedup 1.0000x reference)
import jax
import jax.numpy as jnp
from jax.experimental import pallas as pl


def kernel(data, row_ptr, col_idx, values, bias):
    raise NotImplementedError("write your pallas kernel here")


# speedup vs baseline: 25.5225x; 25.5225x over previous
import functools

import jax
import jax.numpy as jnp
from jax import lax
from jax.experimental import pallas as pl
from jax.experimental.pallas import tpu as pltpu
from jax.experimental.pallas import tpu_sc as plsc

N_ROWS = 16384
N_COLS = 16384
NNZ = 16
BATCH = 64
LANES = 16

NC, NS = 2, 16
NW = NC * NS
ROWS_PER_W = N_ROWS // NW
C = 32
NCH = ROWS_PER_W // C
IPC = C * NNZ
GW = 128
NG = IPC // GW


def _sc_body(data_t, idx2, vals, out, idx_v, vals_v, g_v, out_v, gsem):
    wid = lax.axis_index("s") * NC + lax.axis_index("c")

    @pl.loop(0, NCH)
    def _chunk(g):
        row0 = wid * ROWS_PER_W + g * C
        nnz0 = row0 * NNZ
        ir0 = wid * (ROWS_PER_W * NNZ // GW) + g * NG
        pltpu.sync_copy(idx2.at[pl.ds(ir0, NG)], idx_v)
        pltpu.sync_copy(vals.at[pl.ds(nnz0, IPC)], vals_v)
        descs = [
            pltpu.async_copy(data_t.at[idx_v.at[j]],
                             g_v.at[pl.ds(j * GW, GW)], gsem)
            for j in range(NG)
        ]
        for d in descs:
            d.wait()

        @pl.loop(0, C)
        def _row(r):
            base = r * NNZ
            vrow = vals_v[pl.ds(base, NNZ)]
            vs = [vrow[k] for k in range(NNZ)]
            for c4 in range(BATCH // LANES):
                acc = vs[0] * g_v[base, pl.ds(c4 * LANES, LANES)]
                for k in range(1, NNZ):
                    acc = acc + vs[k] * g_v[base + k, pl.ds(c4 * LANES, LANES)]
                out_v[r, pl.ds(c4 * LANES, LANES)] = acc

        pltpu.sync_copy(out_v, out.at[pl.ds(row0, C)])


_sc_call = pl.kernel(
    _sc_body,
    out_type=jax.ShapeDtypeStruct((N_ROWS, BATCH), jnp.float32),
    mesh=plsc.VectorSubcoreMesh(core_axis_name="c", subcore_axis_name="s",
                                num_cores=NC, num_subcores=NS),
    scratch_types=[
        pltpu.VMEM((NG, GW), jnp.int32),
        pltpu.VMEM((IPC,), jnp.float32),
        pltpu.VMEM((IPC, BATCH), jnp.float32),
        pltpu.VMEM((C, BATCH), jnp.float32),
        pltpu.SemaphoreType.DMA,
    ],
    compiler_params=pltpu.CompilerParams(use_tc_tiling_on_sc=False),
)


def kernel(data, row_ptr, col_idx, values, bias):
    del row_ptr
    data_t = data.T
    idx2 = col_idx.reshape(-1, GW)
    out_t = _sc_call(data_t, idx2, values)
    return out_t.T + bias[None, :]


# speedup vs baseline: 40.7258x; 1.5957x over previous
import jax
import jax.numpy as jnp
from jax import lax
from jax.experimental import pallas as pl
from jax.experimental.pallas import tpu as pltpu
from jax.experimental.pallas import tpu_sc as plsc

N_ROWS = 16384
N_COLS = 16384
NNZ = 16
BATCH = 64
LANES = 16

NC, NS = 2, 16
NW = NC * NS
ROWS_PER_W = N_ROWS // NW
C = 32
NCH = ROWS_PER_W // C
IPC = C * NNZ
GW = 128
NG = IPC // GW
IDXROWS_PER_W = ROWS_PER_W * NNZ // GW


def _sc_body(data_t, idx2, vals, out,
             idx_a, idx_b, vals_a, vals_b, g_a, g_b, out_a, out_b,
             gsem_a, gsem_b, isem_a, isem_b, osem_a, osem_b):
    wid = lax.axis_index("s") * NC + lax.axis_index("c")
    row0_w = wid * ROWS_PER_W
    ir0_w = wid * IDXROWS_PER_W

    idx_bufs = (idx_a, idx_b)
    vals_bufs = (vals_a, vals_b)
    g_bufs = (g_a, g_b)
    out_bufs = (out_a, out_b)
    gsems = (gsem_a, gsem_b)
    isems = (isem_a, isem_b)
    osems = (osem_a, osem_b)

    def idx_copy(g, p):
        return pltpu.make_async_copy(
            idx2.at[pl.ds(ir0_w + g * NG, NG)], idx_bufs[p], isems[p])

    def gather_copies(g, p):
        nnz0 = (row0_w + g * C) * NNZ
        cps = [
            pltpu.make_async_copy(data_t.at[idx_bufs[p].at[j]],
                                  g_bufs[p].at[pl.ds(j * GW, GW)], gsems[p])
            for j in range(NG)
        ]
        cps.append(pltpu.make_async_copy(vals.at[pl.ds(nnz0, IPC)],
                                         vals_bufs[p], gsems[p]))
        return cps

    def out_copy(g, p):
        return pltpu.make_async_copy(
            out_bufs[p], out.at[pl.ds(row0_w + g * C, C)], osems[p])

    pltpu.sync_copy(idx2.at[pl.ds(ir0_w, NG)], idx_bufs[0])
    for cp in gather_copies(0, 0):
        cp.start()
    idx_copy(1, 1).start()

    @pl.loop(0, NCH, step=2)
    def _pair(g0):
        for p in range(2):
            cur = g0 + p
            @pl.when(cur >= 2)
            def _():
                out_copy(cur - 2, p).wait()
            for cp in gather_copies(cur, p):
                cp.wait()
            @pl.when(cur + 1 < NCH)
            def _():
                idx_copy(cur + 1, 1 - p).wait()
                for cp in gather_copies(cur + 1, 1 - p):
                    cp.start()

                @pl.when(cur + 2 < NCH)
                def _():
                    idx_copy(cur + 2, p).start()

            g_v = g_bufs[p]
            vals_v = vals_bufs[p]
            out_v = out_bufs[p]

            @pl.loop(0, C)
            def _row(r):
                base = r * NNZ
                vrow = vals_v[pl.ds(base, NNZ)]
                vs = [vrow[k] for k in range(NNZ)]
                for c4 in range(BATCH // LANES):
                    prods = [
                        vs[k] * g_v[base + k, pl.ds(c4 * LANES, LANES)]
                        for k in range(NNZ)
                    ]
                    while len(prods) > 1:
                        prods = [prods[i] + prods[i + 1]
                                 for i in range(0, len(prods), 2)]
                    out_v[r, pl.ds(c4 * LANES, LANES)] = prods[0]

            out_copy(cur, p).start()

    out_copy(NCH - 2, 0).wait()
    out_copy(NCH - 1, 1).wait()


_sc_call = pl.kernel(
    _sc_body,
    out_type=jax.ShapeDtypeStruct((N_ROWS, BATCH), jnp.float32),
    mesh=plsc.VectorSubcoreMesh(core_axis_name="c", subcore_axis_name="s",
                                num_cores=NC, num_subcores=NS),
    scratch_types=[
        pltpu.VMEM((NG, GW), jnp.int32),
        pltpu.VMEM((NG, GW), jnp.int32),
        pltpu.VMEM((IPC,), jnp.float32),
        pltpu.VMEM((IPC,), jnp.float32),
        pltpu.VMEM((IPC, BATCH), jnp.float32),
        pltpu.VMEM((IPC, BATCH), jnp.float32),
        pltpu.VMEM((C, BATCH), jnp.float32),
        pltpu.VMEM((C, BATCH), jnp.float32),
        pltpu.SemaphoreType.DMA,
        pltpu.SemaphoreType.DMA,
        pltpu.SemaphoreType.DMA,
        pltpu.SemaphoreType.DMA,
        pltpu.SemaphoreType.DMA,
        pltpu.SemaphoreType.DMA,
    ],
    compiler_params=pltpu.CompilerParams(use_tc_tiling_on_sc=False),
)


def kernel(data, row_ptr, col_idx, values, bias):
    del row_ptr
    data_t = data.T
    idx2 = col_idx.reshape(-1, GW)
    out_t = _sc_call(data_t, idx2, values)
    return out_t.T + bias[None, :]
